# baseline (speedup 1.0000x reference)
import functools

import jax
import jax.numpy as jnp
from jax import lax
from jax.experimental import pallas as pl
from jax.experimental.pallas import tpu as pltpu
from jax.experimental.pallas import tpu_sc as plsc

_INFO = plsc.get_sparse_core_info()
_NC = _INFO.num_cores
_NS = _INFO.num_subcores
_NW = _NC * _NS
_LANES = _INFO.num_lanes
_UNROLL = 16


def _make_gather(d: int, vocab: int, n_idx: int):
    rows_per_w = d // _NW
    half = n_idx // 2
    n_groups = half // (_LANES * _UNROLL)
    mesh = plsc.VectorSubcoreMesh(core_axis_name="c", subcore_axis_name="s")

    @functools.partial(
        pl.kernel,
        out_type=jax.ShapeDtypeStruct((2, d, n_idx), jnp.float32),
        mesh=mesh,
        scratch_types=[
            pltpu.VMEM((vocab,), jnp.float32),
            pltpu.VMEM((n_idx,), jnp.int32),
            pltpu.VMEM((n_idx,), jnp.int32),
            pltpu.VMEM((2, half), jnp.float32),
            pltpu.SemaphoreType.DMA,
            pltpu.SemaphoreType.DMA,
            pltpu.SemaphoreType.DMA,
        ],
        compiler_params=pltpu.CompilerParams(needs_layout_passes=False),
    )
    def gather_kernel(idx_text_hbm, idx_tags_hbm, table_t_hbm, out_hbm,
                      row_v, idx_text_v, idx_tags_v, out_v,
                      sem_row, sem_idx, sem_out):
        wid = lax.axis_index("s") * _NC + lax.axis_index("c")
        h_text = pltpu.async_copy(idx_text_hbm, idx_text_v, sem_idx)
        h_tags = pltpu.async_copy(idx_tags_hbm, idx_tags_v, sem_idx)
        h_row = pltpu.async_copy(table_t_hbm.at[wid * rows_per_w], row_v,
                                 sem_row)
        h_text.wait()
        h_tags.wait()

        stores = [None, None]
        for r in range(rows_per_w):
            comp = wid * rows_per_w + r
            h_row.wait()
            for ti, idx_v in ((0, idx_text_v), (1, idx_tags_v)):
                for hi in range(2):
                    buf = (ti * 2 + hi) % 2
                    if stores[buf] is not None:
                        stores[buf].wait()
                    base = hi * half

                    def body(g, _, idx_v=idx_v, buf=buf, base=base):
                        for j in range(_UNROLL):
                            off = (g * _UNROLL + j) * _LANES
                            iv = idx_v[pl.ds(base + off, _LANES)]
                            out_v[buf, pl.ds(off, _LANES)] = (
                                plsc.load_gather(row_v, [iv]))
                        return 0

                    lax.fori_loop(0, n_groups, body, 0)
                    stores[buf] = pltpu.async_copy(
                        out_v.at[buf],
                        out_hbm.at[ti, comp, pl.ds(base, half)],
                        sem_out)
            if r + 1 < rows_per_w:
                h_row = pltpu.async_copy(
                    table_t_hbm.at[comp + 1], row_v, sem_row)
        for h in stores:
            h.wait()

    return gather_kernel


def kernel(text_tokens, predictions, tag_vocab):
    L = text_tokens.shape[0]
    vocab, d = tag_vocab.shape
    slice_tags = predictions[0, -L:]
    out_t = _make_gather(d, vocab, L)(
        text_tokens.astype(jnp.int32),
        slice_tags.astype(jnp.int32),
        tag_vocab.T,
    )
    return jnp.transpose(out_t, (2, 0, 1))


# speedup vs baseline: 1.2284x; 1.2284x over previous
import functools

import jax
import jax.numpy as jnp
from jax import lax
from jax.experimental import pallas as pl
from jax.experimental.pallas import tpu as pltpu
from jax.experimental.pallas import tpu_sc as plsc

_INFO = plsc.get_sparse_core_info()
_NC = _INFO.num_cores
_NS = _INFO.num_subcores
_NW = _NC * _NS
_LANES = _INFO.num_lanes
_UNROLL = 8


def _make_gather(d: int, vocab: int, n_idx: int):
    rows_per_w = d // _NW
    n_groups = n_idx // (_LANES * _UNROLL)
    mesh = plsc.VectorSubcoreMesh(core_axis_name="c", subcore_axis_name="s")

    @functools.partial(
        pl.kernel,
        out_type=jax.ShapeDtypeStruct((2, d, n_idx), jnp.float32),
        mesh=mesh,
        scratch_types=[
            pltpu.VMEM((vocab,), jnp.float32),
            pltpu.VMEM((n_idx,), jnp.int32),
            pltpu.VMEM((n_idx,), jnp.int32),
            pltpu.VMEM((n_idx,), jnp.float32),
            pltpu.SemaphoreType.DMA,
            pltpu.SemaphoreType.DMA,
        ],
        compiler_params=pltpu.CompilerParams(needs_layout_passes=False),
    )
    def gather_kernel(idx_text_hbm, idx_tags_hbm, table_t_hbm, out_hbm,
                      row_v, idx_text_v, idx_tags_v, out_v,
                      sem_row, sem_idx):
        wid = lax.axis_index("s") * _NC + lax.axis_index("c")
        h_text = pltpu.async_copy(idx_text_hbm, idx_text_v, sem_idx)
        h_tags = pltpu.async_copy(idx_tags_hbm, idx_tags_v, sem_idx)
        h_row = pltpu.async_copy(table_t_hbm.at[wid * rows_per_w], row_v,
                                 sem_row)
        h_text.wait()
        h_tags.wait()

        for r in range(rows_per_w):
            comp = wid * rows_per_w + r
            h_row.wait()
            for t, idx_v in ((0, idx_text_v), (1, idx_tags_v)):

                def body(g, _, idx_v=idx_v):
                    for j in range(_UNROLL):
                        off = (g * _UNROLL + j) * _LANES
                        iv = idx_v[pl.ds(off, _LANES)]
                        out_v[pl.ds(off, _LANES)] = plsc.load_gather(
                            row_v, [iv])
                    return 0

                lax.fori_loop(0, n_groups, body, 0)
                pltpu.sync_copy(out_v, out_hbm.at[t, comp])
            if r + 1 < rows_per_w:
                h_row = pltpu.async_copy(
                    table_t_hbm.at[comp + 1], row_v, sem_row)

    return gather_kernel


def kernel(text_tokens, predictions, tag_vocab):
    L = text_tokens.shape[0]
    vocab, d = tag_vocab.shape
    slice_tags = predictions[0, -L:]
    out_t = _make_gather(d, vocab, L)(
        text_tokens.astype(jnp.int32),
        slice_tags.astype(jnp.int32),
        tag_vocab.T,
    )
    return jnp.transpose(out_t, (2, 0, 1))
